# Initial kernel scaffold; baseline (speedup 1.0000x reference)
import jax
import jax.numpy as jnp
from jax.experimental import pallas as pl


def kernel(x, edge_index, batch, W_rel1, b_rel1, W_root1, p1, W_rel2, b_rel2, W_root2, p2, W_rel3, b_rel3, W_root3, p3, W_l1, b_l1, W_l2, b_l2):
    raise NotImplementedError("write your pallas kernel here")


# speedup vs baseline: 12.1195x; 12.1195x over previous
import functools

import jax
import jax.numpy as jnp
from jax import lax
from jax.experimental import pallas as pl
from jax.experimental.pallas import tpu as pltpu
from jax.experimental.pallas import tpu_sc as plsc

N = 10000
E = 320000
H = 128
K1, K2, K3 = 8000, 6400, 5120

NUM_CORES = 2
NUM_SUBCORES = 16
NUM_TILES = NUM_CORES * NUM_SUBCORES
EPT = E // NUM_TILES
CHUNK = 80
NCHUNK = EPT // CHUNK
NPAD = 10240
ROWS_PER_TILE = NPAD // NUM_SUBCORES


def _sc_scatter_body(x_hbm, src_hbm, dst_hbm, zeros_hbm, out_hbm,
                     sidx, didx, rows, acc, sem):
    cid = lax.axis_index("c")
    sid = lax.axis_index("s")
    wid = cid * NUM_SUBCORES + sid

    pltpu.sync_copy(zeros_hbm, acc.at[pl.ds(sid * ROWS_PER_TILE, ROWS_PER_TILE)])
    plsc.subcore_barrier()

    def body(i, carry):
        base = wid * EPT + i * CHUNK
        pltpu.sync_copy(src_hbm.at[pl.ds(base, CHUNK)], sidx)
        pltpu.sync_copy(dst_hbm.at[pl.ds(base, CHUNK)], didx)
        pltpu.async_copy(x_hbm.at[sidx], rows, sem).wait()
        pltpu.sync_copy(rows, acc.at[didx], add=True)
        return carry

    lax.fori_loop(0, NCHUNK, body, 0)
    plsc.subcore_barrier()

    pltpu.sync_copy(
        acc.at[pl.ds(sid * ROWS_PER_TILE, ROWS_PER_TILE)],
        out_hbm.at[pl.ds(cid * NPAD + sid * ROWS_PER_TILE, ROWS_PER_TILE)],
    )


@functools.cache
def _get_sc_scatter():
    return pl.kernel(
        _sc_scatter_body,
        out_type=jax.ShapeDtypeStruct((NUM_CORES * NPAD, H), jnp.float32),
        mesh=plsc.VectorSubcoreMesh(core_axis_name="c", subcore_axis_name="s"),
        scratch_types=[
            pltpu.VMEM((CHUNK,), jnp.int32),
            pltpu.VMEM((CHUNK,), jnp.int32),
            pltpu.VMEM((CHUNK, H), jnp.float32),
            pltpu.VMEM_SHARED((NPAD, H), jnp.float32),
            pltpu.SemaphoreType.DMA,
        ],
    )


def _sc_scatter(*args):
    return _get_sc_scatter()(*args)


def _sortable(v):
    bits = lax.bitcast_convert_type(v, jnp.int32)
    return jnp.where(bits >= 0, bits, bits ^ jnp.int32(0x7FFFFFFF))


def _topk_keep(keys, k):
    i32 = jnp.int32
    shape = keys[0].shape
    idx = (lax.broadcasted_iota(i32, shape, 0) * shape[1]
           + lax.broadcasted_iota(i32, shape, 1))

    def cnt(pred):
        return jnp.sum(pred.astype(i32))

    sure = jnp.zeros(shape, jnp.bool_)
    cand = jnp.ones(shape, jnp.bool_)
    remaining = i32(k)

    for key in keys:
        khi = lax.shift_right_arithmetic(key, 16)
        klo = lax.bitwise_and(key, i32(0xFFFF))

        def hbody(_, c, khi=khi, cand=cand, remaining=remaining):
            lo, hi = c
            mid = lax.shift_right_arithmetic(lo + hi, 1)
            ge = cnt(cand & (khi >= mid)) >= remaining
            return (jnp.where(ge, mid, lo), jnp.where(ge, hi, mid))
        hstar, _ = lax.fori_loop(0, 17, hbody, (i32(-32768), i32(32768)))

        on_h = cand & (khi == hstar)
        rem2 = remaining - cnt(cand & (khi > hstar))

        def lbody(_, c, klo=klo, on_h=on_h, rem2=rem2):
            lo, hi = c
            mid = lax.shift_right_arithmetic(lo + hi, 1)
            ge = cnt(on_h & (klo >= mid)) >= rem2
            return (jnp.where(ge, mid, lo), jnp.where(ge, hi, mid))
        lstar, _ = lax.fori_loop(0, 17, lbody, (i32(0), i32(65536)))

        above_l = on_h & (klo > lstar)
        sure = sure | (cand & (khi > hstar)) | above_l
        remaining = rem2 - cnt(above_l)
        cand = on_h & (klo == lstar)

    def cbody(_, c, cand=cand, remaining=remaining):
        lo, hi = c
        mid = lax.shift_right_arithmetic(lo + hi, 1)
        ge = cnt(cand & (idx < mid)) >= remaining
        return (jnp.where(ge, lo, mid), jnp.where(ge, mid, hi))
    _, cstar = lax.fori_loop(0, 14, cbody, (i32(0), i32(NPAD)))

    return sure | (cand & (idx < cstar))


BLK = 2000
GRID = N // BLK
NLANE = NPAD // 8


def _conv_body(agg0_ref, agg1_ref, x_ref, wrel_ref, brel_ref, wroot_ref,
               p_ref, h_ref, arg_ref):
    agg = agg0_ref[...] + agg1_ref[...]
    h = (jnp.dot(agg, wrel_ref[...], preferred_element_type=jnp.float32)
         + brel_ref[...]
         + jnp.dot(x_ref[...], wroot_ref[...],
                   preferred_element_type=jnp.float32))
    h = jnp.maximum(h, 0.0)
    p = p_ref[...]
    inv_norm = lax.rsqrt(jnp.sum(p * p))
    h_ref[...] = h
    arg_ref[...] = jnp.dot(h, p, preferred_element_type=jnp.float32) * inv_norm


def _tc_conv(agg0, agg1, x, wrel, brel, wroot, p):
    row = lambda i: (i, 0)
    full = lambda i: (0, 0)
    return pl.pallas_call(
        _conv_body,
        grid=(GRID,),
        in_specs=[
            pl.BlockSpec((BLK, H), row),
            pl.BlockSpec((BLK, H), row),
            pl.BlockSpec((BLK, H), row),
            pl.BlockSpec((H, H), full),
            pl.BlockSpec((1, H), full),
            pl.BlockSpec((H, H), full),
            pl.BlockSpec((H, 1), full),
        ],
        out_specs=[
            pl.BlockSpec((BLK, H), row),
            pl.BlockSpec((BLK, 1), row),
        ],
        out_shape=[
            jax.ShapeDtypeStruct((N, H), jnp.float32),
            jax.ShapeDtypeStruct((N, 1), jnp.float32),
        ],
    )(agg0, agg1, x, wrel, brel.reshape(1, H), wroot, p.reshape(H, 1))


def _select_body(*refs, k, n_hist):
    arg_ref, alive_ref = refs[:2]
    hist_refs = refs[2:2 + n_hist]
    keep_ref, factor_ref, score_ref = refs[2 + n_hist:]
    alive = alive_ref[...]
    score = jnp.tanh(arg_ref[...])
    masked = jnp.where(alive > 0.0, score, -jnp.inf)
    keys = [_sortable(masked)] + [_sortable(r[...]) for r in hist_refs]
    keep = _topk_keep(keys, k)
    keep_ref[...] = keep.astype(jnp.float32)
    factor_ref[...] = jnp.where(keep, score, 0.0)
    score_ref[...] = score


def _tc_select(argc, alivec, hist, k):
    return pl.pallas_call(
        functools.partial(_select_body, k=k, n_hist=len(hist)),
        out_shape=[
            jax.ShapeDtypeStruct((8, NLANE), jnp.float32),
            jax.ShapeDtypeStruct((8, NLANE), jnp.float32),
            jax.ShapeDtypeStruct((8, NLANE), jnp.float32),
        ],
    )(argc, alivec, *hist)


def _scale_body(h_ref, f_ref, xn_ref):
    xn_ref[...] = h_ref[...] * f_ref[...]


def _tc_scale(h, factor):
    row = lambda i: (i, 0)
    return pl.pallas_call(
        _scale_body,
        grid=(GRID,),
        in_specs=[
            pl.BlockSpec((BLK, H), row),
            pl.BlockSpec((BLK, 1), row),
        ],
        out_specs=pl.BlockSpec((BLK, H), row),
        out_shape=jax.ShapeDtypeStruct((N, H), jnp.float32),
    )(h, factor)


def _head_body(h_ref, f_ref, wl1_ref, bl1_ref, wl2_ref, bl2_ref, out_ref,
               acc_ref):
    i = pl.program_id(0)

    @pl.when(i == 0)
    def _():
        acc_ref[...] = jnp.zeros_like(acc_ref)

    part = jnp.sum(h_ref[...] * f_ref[...], axis=0, keepdims=True)
    acc_ref[...] += part

    @pl.when(i == GRID - 1)
    def _():
        g = acc_ref[...] * (1.0 / K3)
        z = jnp.maximum(
            jnp.dot(g, wl1_ref[...], preferred_element_type=jnp.float32)
            + bl1_ref[...], 0.0)
        logits = (jnp.dot(z, wl2_ref[...], preferred_element_type=jnp.float32)
                  + bl2_ref[...])
        m = jnp.max(logits, axis=-1, keepdims=True)
        lse = jnp.log(jnp.sum(jnp.exp(logits - m), axis=-1,
                              keepdims=True)) + m
        out_ref[...] = logits - lse


def _tc_head(h, factor, wl1, bl1, wl2, bl2):
    row = lambda i: (i, 0)
    full = lambda i: (0, 0)
    return pl.pallas_call(
        _head_body,
        grid=(GRID,),
        in_specs=[
            pl.BlockSpec((BLK, H), row),
            pl.BlockSpec((BLK, 1), row),
            pl.BlockSpec((H, 64), full),
            pl.BlockSpec((1, 64), full),
            pl.BlockSpec((64, 16), full),
            pl.BlockSpec((1, 16), full),
        ],
        out_specs=pl.BlockSpec((1, 16), full),
        out_shape=jax.ShapeDtypeStruct((1, 16), jnp.float32),
        scratch_shapes=[pltpu.VMEM((1, H), jnp.float32)],
    )(h, factor, wl1, bl1.reshape(1, 64), wl2, bl2.reshape(1, 16))


def _compact(col):
    flat = col.reshape(N)
    return jnp.pad(flat, (0, NPAD - N),
                   constant_values=-jnp.inf).reshape(8, NLANE)


def _uncompact(c):
    return c.reshape(NPAD, 1)[:N]


def kernel(x, edge_index, batch, W_rel1, b_rel1, W_root1, p1, W_rel2, b_rel2,
           W_root2, p2, W_rel3, b_rel3, W_root3, p3, W_l1, b_l1, W_l2, b_l2):
    src = edge_index[0]
    dst = edge_index[1]
    zeros = jnp.zeros((ROWS_PER_TILE, H), jnp.float32)
    alivec = (jnp.arange(NPAD, dtype=jnp.int32) < N).astype(
        jnp.float32).reshape(8, NLANE)

    rounds = [
        (W_rel1, b_rel1, W_root1, p1, K1),
        (W_rel2, b_rel2, W_root2, p2, K2),
        (W_rel3, b_rel3, W_root3, p3, K3),
    ]
    hist = []
    xr = x
    for i, (wrel, brel, wroot, p, k) in enumerate(rounds):
        aggp = _sc_scatter(xr, src, dst, zeros)
        agg0 = lax.slice(aggp, (0, 0), (N, H))
        agg1 = lax.slice(aggp, (NPAD, 0), (NPAD + N, H))
        h, arg = _tc_conv(agg0, agg1, xr, wrel, brel, wroot, p)
        argc = _compact(arg)
        keepc, factorc, scorec = _tc_select(argc, alivec, tuple(hist[::-1]), k)
        factor = _uncompact(factorc)
        if i < 2:
            xr = _tc_scale(h, factor)
            alivec = keepc
            hist.append(scorec)
    return _tc_head(h, factor, W_l1, b_l1, W_l2, b_l2)
